# baseline (speedup 1.0000x reference)
import functools
import math

import jax
import jax.numpy as jnp
from jax import lax
from jax.experimental import pallas as pl
from jax.experimental.pallas import tpu as pltpu
from jax.experimental.pallas import tpu_sc as plsc

D_MODEL = 64
SCALE = math.sqrt(D_MODEL)

NUM_CORES = 2
NUM_SUBCORES = 16
NUM_WORKERS = NUM_CORES * NUM_SUBCORES

ROWS_PER_GATHER = 128


def _emb_body(x_hbm, table_hbm, out_hbm, idx_v, buf_v, sem, *, chunks_per_worker):
    wid = lax.axis_index("s") * NUM_CORES + lax.axis_index("c")
    chunk0 = wid * chunks_per_worker
    row0 = chunk0 * ROWS_PER_GATHER

    pltpu.sync_copy(x_hbm.at[pl.ds(chunk0, chunks_per_worker)], idx_v)

    def chunk_body(j, carry):
        pltpu.async_copy(table_hbm.at[idx_v.at[j]], buf_v, sem).wait()

        def scale_row(r, c2):
            for c in range(D_MODEL // 16):
                sl = pl.ds(c * 16, 16)
                buf_v[r, sl] = buf_v[r, sl] * SCALE
            return c2

        lax.fori_loop(0, ROWS_PER_GATHER, scale_row, 0)

        pltpu.sync_copy(
            buf_v, out_hbm.at[pl.ds(row0 + j * ROWS_PER_GATHER, ROWS_PER_GATHER)]
        )
        return carry

    lax.fori_loop(0, chunks_per_worker, chunk_body, 0)


@jax.jit
def kernel(x, table):
    orig_shape = x.shape
    b_total = x.size
    assert b_total % (NUM_WORKERS * ROWS_PER_GATHER) == 0
    chunks_per_worker = b_total // (NUM_WORKERS * ROWS_PER_GATHER)

    x2d = x.reshape(b_total // ROWS_PER_GATHER, ROWS_PER_GATHER)

    mesh = plsc.VectorSubcoreMesh(core_axis_name="c", subcore_axis_name="s")
    out = pl.kernel(
        functools.partial(_emb_body, chunks_per_worker=chunks_per_worker),
        mesh=mesh,
        compiler_params=pltpu.CompilerParams(use_tc_tiling_on_sc=False),
        out_type=jax.ShapeDtypeStruct((b_total, D_MODEL), jnp.float32),
        scratch_types=[
            pltpu.VMEM((chunks_per_worker, ROWS_PER_GATHER), jnp.int32),
            pltpu.VMEM((ROWS_PER_GATHER, D_MODEL), jnp.float32),
            pltpu.SemaphoreType.DMA,
        ],
    )(x2d, table)
    return out.reshape(*orig_shape, D_MODEL)


# speedup vs baseline: 1.1080x; 1.1080x over previous
import functools
import math

import jax
import jax.numpy as jnp
from jax import lax
from jax.experimental import pallas as pl
from jax.experimental.pallas import tpu as pltpu
from jax.experimental.pallas import tpu_sc as plsc

D_MODEL = 64
D_PAD = 128
SCALE = math.sqrt(D_MODEL)

NUM_CORES = 2
NUM_SUBCORES = 16
NUM_WORKERS = NUM_CORES * NUM_SUBCORES

SEQ = 200
SPLIT_A = 104
SPLIT_B = SEQ - SPLIT_A


def _emb_body(x_hbm, table_hbm, out_hbm, idx_v, buf_v, cbuf_v, sem, *, sents_per_worker):
    wid = lax.axis_index("s") * NUM_CORES + lax.axis_index("c")
    sent0 = wid * sents_per_worker

    def sent_body(j, carry):
        sent = sent0 + j
        pltpu.sync_copy(x_hbm.at[sent], idx_v)
        cp0 = pltpu.async_copy(
            table_hbm.at[idx_v.at[pl.ds(0, SPLIT_A)]],
            buf_v.at[pl.ds(0, SPLIT_A)],
            sem,
        )
        cp1 = pltpu.async_copy(
            table_hbm.at[idx_v.at[pl.ds(SPLIT_A, SPLIT_B)]],
            buf_v.at[pl.ds(SPLIT_A, SPLIT_B)],
            sem,
        )
        cp0.wait()
        cp1.wait()

        def scale_row(r, c2):
            for c in range(D_MODEL // 16):
                cbuf_v[r, pl.ds(c * 16, 16)] = buf_v[r, pl.ds(c * 16, 16)] * SCALE
            return c2

        lax.fori_loop(0, SEQ, scale_row, 0)

        pltpu.sync_copy(cbuf_v, out_hbm.at[sent])
        return carry

    lax.fori_loop(0, sents_per_worker, sent_body, 0)


@jax.jit
def kernel(x, table):
    n_sent, seq = x.shape
    assert seq == SEQ and n_sent % NUM_WORKERS == 0
    sents_per_worker = n_sent // NUM_WORKERS

    table_wide = jnp.pad(table, ((0, 0), (0, D_PAD - D_MODEL)))

    mesh = plsc.VectorSubcoreMesh(core_axis_name="c", subcore_axis_name="s")
    out = pl.kernel(
        functools.partial(_emb_body, sents_per_worker=sents_per_worker),
        mesh=mesh,
        out_type=jax.ShapeDtypeStruct((n_sent, SEQ, D_MODEL), jnp.float32),
        scratch_types=[
            pltpu.VMEM((SEQ,), jnp.int32),
            pltpu.VMEM((SEQ, D_PAD), jnp.float32),
            pltpu.VMEM((SEQ, D_MODEL), jnp.float32),
            pltpu.SemaphoreType.DMA,
        ],
    )(x, table_wide)
    return out


# speedup vs baseline: 1.3383x; 1.2078x over previous
import functools
import math

import jax
import jax.numpy as jnp
from jax import lax
from jax.experimental import pallas as pl
from jax.experimental.pallas import tpu as pltpu
from jax.experimental.pallas import tpu_sc as plsc

D_MODEL = 64
D_PAD = 128
SCALE = math.sqrt(D_MODEL)

NUM_CORES = 2
NUM_SUBCORES = 16
NUM_WORKERS = NUM_CORES * NUM_SUBCORES

SEQ = 200
SPLIT_A = 104
SPLIT_B = SEQ - SPLIT_A


def _emb_body(
    x_hbm,
    table_hbm,
    out_hbm,
    idx_v,
    g0,
    g1,
    o0,
    o1,
    gs0,
    gs1,
    os0,
    os1,
    *,
    sents_per_worker,
):
    wid = lax.axis_index("s") * NUM_CORES + lax.axis_index("c")
    sent0 = wid * sents_per_worker
    idx0 = wid * (sents_per_worker * SEQ)

    def fire_gather(s_local, gbuf, gsem):
        base = s_local * SEQ
        pltpu.async_copy(
            table_hbm.at[idx_v.at[pl.ds(base, SPLIT_A)]],
            gbuf.at[pl.ds(0, SPLIT_A)],
            gsem,
        )
        pltpu.async_copy(
            table_hbm.at[idx_v.at[pl.ds(base + SPLIT_A, SPLIT_B)]],
            gbuf.at[pl.ds(SPLIT_A, SPLIT_B)],
            gsem,
        )

    def wait_gather(s_local, gbuf, gsem):
        base = s_local * SEQ
        pltpu.make_async_copy(
            table_hbm.at[idx_v.at[pl.ds(base, SPLIT_A)]],
            gbuf.at[pl.ds(0, SPLIT_A)],
            gsem,
        ).wait()
        pltpu.make_async_copy(
            table_hbm.at[idx_v.at[pl.ds(base + SPLIT_A, SPLIT_B)]],
            gbuf.at[pl.ds(SPLIT_A, SPLIT_B)],
            gsem,
        ).wait()

    pltpu.sync_copy(x_hbm.at[pl.ds(idx0, sents_per_worker * SEQ)], idx_v)

    fire_gather(0, g0, gs0)

    def pair_body(p, carry):
        for b in (0, 1):
            s = 2 * p + b
            gbuf, gsem = (g0, gs0) if b == 0 else (g1, gs1)
            nbuf, nsem = (g1, gs1) if b == 0 else (g0, gs0)
            obuf, osem = (o0, os0) if b == 0 else (o1, os1)

            @pl.when(s + 1 < sents_per_worker)
            def _():
                fire_gather(s + 1, nbuf, nsem)

            wait_gather(s, gbuf, gsem)

            @pl.when(s >= 2)
            def _():
                pltpu.make_async_copy(obuf, out_hbm.at[sent0 + s - 2], osem).wait()

            def scale_row(r, c2):
                for c in range(D_MODEL // 16):
                    sl = pl.ds(c * 16, 16)
                    obuf[r, sl] = gbuf[r, sl] * SCALE
                return c2

            lax.fori_loop(0, SEQ, scale_row, 0)

            pltpu.async_copy(obuf, out_hbm.at[sent0 + s], osem)
        return carry

    lax.fori_loop(0, sents_per_worker // 2, pair_body, 0)

    pltpu.make_async_copy(
        o0, out_hbm.at[sent0 + sents_per_worker - 2], os0
    ).wait()
    pltpu.make_async_copy(
        o1, out_hbm.at[sent0 + sents_per_worker - 1], os1
    ).wait()


@jax.jit
def kernel(x, table):
    n_sent, seq = x.shape
    assert seq == SEQ and n_sent % NUM_WORKERS == 0
    sents_per_worker = n_sent // NUM_WORKERS

    table_wide = jnp.pad(table, ((0, 0), (0, D_PAD - D_MODEL)))
    x_flat = x.reshape(-1)

    mesh = plsc.VectorSubcoreMesh(core_axis_name="c", subcore_axis_name="s")
    out = pl.kernel(
        functools.partial(_emb_body, sents_per_worker=sents_per_worker),
        mesh=mesh,
        out_type=jax.ShapeDtypeStruct((n_sent, SEQ, D_MODEL), jnp.float32),
        scratch_types=[
            pltpu.VMEM((sents_per_worker * SEQ,), jnp.int32),
            pltpu.VMEM((SEQ, D_PAD), jnp.float32),
            pltpu.VMEM((SEQ, D_PAD), jnp.float32),
            pltpu.VMEM((SEQ, D_MODEL), jnp.float32),
            pltpu.VMEM((SEQ, D_MODEL), jnp.float32),
            pltpu.SemaphoreType.DMA,
            pltpu.SemaphoreType.DMA,
            pltpu.SemaphoreType.DMA,
            pltpu.SemaphoreType.DMA,
        ],
    )(x_flat, table_wide)
    return out
